# baseline (speedup 1.0000x reference)
import functools

import jax
import jax.numpy as jnp
from jax import lax
from jax.experimental import pallas as pl
from jax.experimental.pallas import tpu as pltpu
from jax.experimental.pallas import tpu_sc as plsc

NC = 2
NS = 16
L = 16
NW = NC * NS
B = 16384
BPW = B // NW
CHUNK = 128
NCHUNK = BPW // CHUNK
D = 32
LINE = 4 * D
GPC = CHUNK // L


def _sc_scores(ri, co, app2, ent2):
  mesh = plsc.VectorSubcoreMesh(
      core_axis_name="c", subcore_axis_name="s", num_cores=NC, num_subcores=NS)

  @functools.partial(
      pl.kernel,
      out_type=jax.ShapeDtypeStruct((2, NW, BPW), jnp.float32),
      mesh=mesh,
      compiler_params=pltpu.CompilerParams(needs_layout_passes=False),
      scratch_types=[
          pltpu.VMEM((BPW,), jnp.int32),
          pltpu.VMEM((BPW,), jnp.int32),
          pltpu.VMEM((BPW,), jnp.int32),
          pltpu.VMEM((BPW,), jnp.int32),
          pltpu.VMEM((BPW,), jnp.int32),
          pltpu.VMEM((BPW,), jnp.int32),
          pltpu.VMEM((BPW,), jnp.int32),
          pltpu.VMEM((BPW,), jnp.int32),
          pltpu.VMEM((CHUNK, LINE), jnp.float32),
          pltpu.VMEM((CHUNK, LINE), jnp.float32),
          pltpu.VMEM((CHUNK, LINE), jnp.float32),
          pltpu.VMEM((CHUNK, LINE), jnp.float32),
          pltpu.VMEM((BPW,), jnp.float32),
          pltpu.VMEM((BPW,), jnp.float32),
          pltpu.SemaphoreType.DMA,
      ],
  )
  def k(ri_pa, ri_pe, ri_na, ri_ne, co_pa, co_pe, co_na, co_ne,
        app_t, ent_t, out_h,
        pa_i, pe_i, na_i, ne_i, pa_c, pe_c, na_c, ne_c,
        pa_r, pe_r, na_r, ne_r, s_pos, s_neg, sem):
    wid = lax.axis_index("s") * NC + lax.axis_index("c")

    pltpu.sync_copy(ri_pa.at[wid], pa_i)
    pltpu.sync_copy(ri_pe.at[wid], pe_i)
    pltpu.sync_copy(ri_na.at[wid], na_i)
    pltpu.sync_copy(ri_ne.at[wid], ne_i)
    pltpu.sync_copy(co_pa.at[wid], pa_c)
    pltpu.sync_copy(co_pe.at[wid], pe_c)
    pltpu.sync_copy(co_na.at[wid], na_c)
    pltpu.sync_copy(co_ne.at[wid], ne_c)

    lane = lax.iota(jnp.int32, L)

    for c in range(NCHUNK):
      isl = pl.ds(c * CHUNK, CHUNK)
      copies = [
          pltpu.async_copy(app_t.at[pa_i.at[isl]], pa_r, sem),
          pltpu.async_copy(ent_t.at[pe_i.at[isl]], pe_r, sem),
          pltpu.async_copy(app_t.at[na_i.at[isl]], na_r, sem),
          pltpu.async_copy(ent_t.at[ne_i.at[isl]], ne_r, sem),
      ]
      for cp in copies:
        cp.wait()

      def g_body(gl, _, c=c):
        rows = gl * L + lane
        off = c * CHUNK + gl * L
        ca = pa_c[pl.ds(off, L)]
        ce = pe_c[pl.ds(off, L)]
        cna = na_c[pl.ds(off, L)]
        cne = ne_c[pl.ds(off, L)]
        accp = jnp.zeros((L,), jnp.float32)
        accn = jnp.zeros((L,), jnp.float32)
        for d in range(D):
          accp += (plsc.load_gather(pa_r, [rows, ca + d])
                   * plsc.load_gather(pe_r, [rows, ce + d]))
          accn += (plsc.load_gather(na_r, [rows, cna + d])
                   * plsc.load_gather(ne_r, [rows, cne + d]))
        s_pos[pl.ds(off, L)] = accp
        s_neg[pl.ds(off, L)] = accn
        return 0

      lax.fori_loop(0, GPC, g_body, 0)

    pltpu.sync_copy(s_pos, out_h.at[0, wid])
    pltpu.sync_copy(s_neg, out_h.at[1, wid])

  return k(ri[0], ri[1], ri[2], ri[3], co[0], co[1], co[2], co[3], app2, ent2)


def _tc_reduce(scores):
  def body(x_ref, o_ref):
    x = x_ref[...]
    row = lax.broadcasted_iota(jnp.int32, x.shape, 0)
    s = jnp.where(row < 128, x, -x)
    ls = jnp.minimum(s, 0.0) - jnp.log1p(jnp.exp(-jnp.abs(s)))
    o_ref[0, 0] = -jnp.sum(ls)

  out = pl.pallas_call(
      body,
      out_shape=jax.ShapeDtypeStruct((1, 1), jnp.float32),
      out_specs=pl.BlockSpec(memory_space=pltpu.SMEM),
  )(scores)
  return out[0, 0]


def kernel(pos_app, pos_entity, neg_app, neg_entity, app_emb, entity_emb):
  idx = [x.astype(jnp.int32) for x in (pos_app, pos_entity, neg_app, neg_entity)]
  ri = [(x >> 2).reshape(NW, BPW) for x in idx]
  co = [((x & 3) * D).reshape(NW, BPW) for x in idx]
  app2 = app_emb.reshape(app_emb.shape[0] // 4, LINE)
  ent2 = entity_emb.reshape(entity_emb.shape[0] // 4, LINE)
  scores = _sc_scores(ri, co, app2, ent2)
  return _tc_reduce(scores.reshape(2 * B // 128, 128))
